# baseline (speedup 1.0000x reference)
import functools

import jax
import jax.numpy as jnp
from jax import lax
from jax.experimental import pallas as pl
from jax.experimental.pallas import tpu as pltpu
from jax.experimental.pallas import tpu_sc as plsc

B = 16384
D = 32
WD = 1e-4

_NC = 2
_NS = 16
_NW = _NC * _NS
_BPW = B // _NW
_CH = 128
_NCHUNK = _BPW // _CH


def _sc_gather(u, i, j, W_mlp, H_mlp, W_mf, H_mf):
    mesh = plsc.VectorSubcoreMesh(core_axis_name="c", subcore_axis_name="s")
    out_t = tuple(jax.ShapeDtypeStruct((B, D), jnp.float32) for _ in range(6))

    @functools.partial(
        pl.kernel,
        mesh=mesh,
        out_type=out_t,
        compiler_params=pltpu.CompilerParams(use_tc_tiling_on_sc=False),
        scratch_types=[
            pltpu.VMEM((_NCHUNK, _CH), jnp.int32),
            pltpu.VMEM((_NCHUNK, _CH), jnp.int32),
            pltpu.VMEM((_NCHUNK, _CH), jnp.int32),
            pltpu.VMEM((_BPW, D), jnp.float32),
            pltpu.VMEM((_BPW, D), jnp.float32),
            pltpu.VMEM((_BPW, D), jnp.float32),
            pltpu.VMEM((_BPW, D), jnp.float32),
            pltpu.VMEM((_BPW, D), jnp.float32),
            pltpu.VMEM((_BPW, D), jnp.float32),
            pltpu.SemaphoreType.DMA,
        ],
    )
    def gather_kernel(u_hbm, i_hbm, j_hbm, wmlp, hmlp, wmf, hmf,
                      o_umlp, o_imlp, o_jmlp, o_umf, o_imf, o_jmf,
                      uidx, iidx, jidx, r0, r1, r2, r3, r4, r5, sem):
        wid = lax.axis_index("s") * _NC + lax.axis_index("c")
        base = wid * _BPW
        for c in range(_NCHUNK):
            src = pl.ds(base + c * _CH, _CH)
            pltpu.sync_copy(u_hbm.at[src], uidx.at[c])
            pltpu.sync_copy(i_hbm.at[src], iidx.at[c])
            pltpu.sync_copy(j_hbm.at[src], jidx.at[c])
        copies = []
        for c in range(_NCHUNK):
            sl = pl.ds(c * _CH, _CH)
            copies.append(pltpu.async_copy(wmlp.at[uidx.at[c]], r0.at[sl], sem))
            copies.append(pltpu.async_copy(hmlp.at[iidx.at[c]], r1.at[sl], sem))
            copies.append(pltpu.async_copy(hmlp.at[jidx.at[c]], r2.at[sl], sem))
            copies.append(pltpu.async_copy(wmf.at[uidx.at[c]], r3.at[sl], sem))
            copies.append(pltpu.async_copy(hmf.at[iidx.at[c]], r4.at[sl], sem))
            copies.append(pltpu.async_copy(hmf.at[jidx.at[c]], r5.at[sl], sem))
        for cp in copies:
            cp.wait()
        dst = pl.ds(base, _BPW)
        pltpu.sync_copy(r0, o_umlp.at[dst])
        pltpu.sync_copy(r1, o_imlp.at[dst])
        pltpu.sync_copy(r2, o_jmlp.at[dst])
        pltpu.sync_copy(r3, o_umf.at[dst])
        pltpu.sync_copy(r4, o_imf.at[dst])
        pltpu.sync_copy(r5, o_jmf.at[dst])

    return gather_kernel(u, i, j, W_mlp, H_mlp, W_mf, H_mf)


_BLK = 2048


def _tc_head_kernel(ue_mlp_ref, ie_mlp_ref, je_mlp_ref,
                    ue_mf_ref, ie_mf_ref, je_mf_ref,
                    fc0t_ref, fc0b_ref, fc1t_ref, fc1b_ref,
                    afft_ref, affb_ref, out_ref):
    ue = ue_mlp_ref[...]
    ie = ie_mlp_ref[...]
    je = je_mlp_ref[...]
    uef = ue_mf_ref[...]
    ief = ie_mf_ref[...]
    jef = je_mf_ref[...]
    fc0t = fc0t_ref[...]
    a0u = fc0t[:D]
    a0i = fc0t[D:]
    fc1t = fc1t_ref[...]
    afft = afft_ref[...]
    b0 = fc0b_ref[...]
    b1 = fc1b_ref[...]

    def head(item_mlp, item_mf):
        h0 = jnp.maximum(
            jnp.dot(ue, a0u, preferred_element_type=jnp.float32)
            + jnp.dot(item_mlp, a0i, preferred_element_type=jnp.float32)
            + b0, 0.0)
        h1 = jnp.maximum(
            jnp.dot(h0, fc1t, preferred_element_type=jnp.float32) + b1, 0.0)
        mf = uef * item_mf
        logit = (jnp.dot(h1, afft[:16], preferred_element_type=jnp.float32)
                 + jnp.dot(mf, afft[16:], preferred_element_type=jnp.float32))
        return logit[:, 0] + affb_ref[0, 0]

    x = head(ie, ief) - head(je, jef)
    neg_log_prob = jnp.maximum(-x, 0.0) + jnp.log1p(jnp.exp(-jnp.abs(x)))

    def nrm(a):
        return jnp.sqrt(jnp.sum(a * a, axis=1))

    reg = WD * (nrm(ue) + nrm(uef) + nrm(ie) + nrm(ief) + nrm(je) + nrm(jef))
    out_ref[...] = neg_log_prob + reg


def _tc_head(ue_mlp, ie_mlp, je_mlp, ue_mf, ie_mf, je_mf,
             fc0t, fc0b, fc1t, fc1b, afft, affb):
    row_spec = pl.BlockSpec((_BLK, D), lambda b: (b, 0))

    def full(shape):
        return pl.BlockSpec(shape, lambda b, _n=len(shape): (0,) * _n)

    return pl.pallas_call(
        _tc_head_kernel,
        grid=(B // _BLK,),
        in_specs=[row_spec] * 6 + [
            full((64, D)), full((1, D)), full((D, 16)), full((1, 16)),
            full((48, 1)), full((1, 1)),
        ],
        out_specs=pl.BlockSpec((_BLK,), lambda b: (b,)),
        out_shape=jax.ShapeDtypeStruct((B,), jnp.float32),
    )(ue_mlp, ie_mlp, je_mlp, ue_mf, ie_mf, je_mf,
      fc0t, fc0b, fc1t, fc1b, afft, affb)


def kernel(u, i, j, W_mlp, H_mlp, W_mf, H_mf,
           fc0_w, fc0_b, fc1_w, fc1_b, aff_w, aff_b):
    ue_mlp, ie_mlp, je_mlp, ue_mf, ie_mf, je_mf = _sc_gather(
        u, i, j, W_mlp, H_mlp, W_mf, H_mf)
    return _tc_head(
        ue_mlp, ie_mlp, je_mlp, ue_mf, ie_mf, je_mf,
        fc0_w.T, fc0_b.reshape(1, D),
        fc1_w.T, fc1_b.reshape(1, 16),
        aff_w.T, aff_b.reshape(1, 1))
